# baseline (speedup 1.0000x reference)
import functools

import jax
import jax.numpy as jnp
import numpy as np
from jax import lax
from jax.experimental import pallas as pl
from jax.experimental.pallas import tpu as pltpu
from jax.experimental.pallas import tpu_sc as plsc

_MIN_RES = 16
_MAX_RES = 512
_NUM_LOD = 16
_CODEBOOK = 2 ** 16
_FEAT = 3
_bexp = np.exp((np.log(_MAX_RES) - np.log(_MIN_RES)) / (_NUM_LOD - 1))
_LODS = [int(1 + np.floor(_MIN_RES * _bexp ** l)) for l in range(_NUM_LOD)]
_SIZES = [min(l * l, _CODEBOOK) for l in _LODS]
_PRIME = np.uint32(2654435761)

_N_SMALL = 6
_SMALL_SIZES = _SIZES[:_N_SMALL]
_SMALL_BASES = np.concatenate(([0], np.cumsum(_SMALL_SIZES)))[:-1].tolist()
_SMALL_TOTAL = int(np.sum(_SMALL_SIZES))
_SMALL_PAD = ((_SMALL_TOTAL + 63) // 64) * 64

_CORE_SMALL = ([0, 1, 2], [3, 4, 5])
_CORE_STREAM = ([6, 7, 11, 12, 13], [8, 9, 10, 14, 15])
_N_STREAM = 5
_SPM_ROWS = max(sum(_SIZES[l] for l in s) for s in _CORE_STREAM)
_SPM_PAD = ((_SPM_ROWS + 127) // 128) * 128


def _stream_bases(core):
    bases, acc = [], 0
    for l in _CORE_STREAM[core]:
        bases.append(acc)
        acc += _SIZES[l]
    return bases


_B = 128
_G = _B // 16


def _corner_indices(cx, cy, res, size, base):
    if res * res <= size:
        i00 = cy * res + (cx + base)
        i10 = i00 + 1
        i01 = i00 + res
        i11 = i01 + 1
    else:
        ux = cx.astype(jnp.uint32)
        uy = cy.astype(jnp.uint32)
        ux1 = ux + jnp.uint32(1)
        hy0 = uy * _PRIME
        hy1 = hy0 + _PRIME
        m = jnp.uint32(size - 1)
        i00 = (( ux ^ hy0) & m).astype(jnp.int32) + base
        i10 = ((ux1 ^ hy0) & m).astype(jnp.int32) + base
        i01 = (( ux ^ hy1) & m).astype(jnp.int32) + base
        i11 = ((ux1 ^ hy1) & m).astype(jnp.int32) + base
    return i00, i10, i01, i11


def _coords(xs, ys, res):
    sx = xs * np.float32(res - 1)
    sy = ys * np.float32(res - 1)
    cx = jnp.minimum(sx.astype(jnp.int32), res - 2)
    cy = jnp.minimum(sy.astype(jnp.int32), res - 2)
    fx = sx - cx.astype(jnp.float32)
    fy = sy - cy.astype(jnp.float32)
    gx = 1.0 - fx
    gy = 1.0 - fy
    w = (gx * gy, fx * gy, gx * fy, fx * fy)
    return cx, cy, w


def _make_sc_kernel(npts_pad):
    info = plsc.get_sparse_core_info()
    nc, ns = info.num_cores, info.num_subcores
    pts_per_tile = npts_pad // ns
    chunks = pts_per_tile // _B
    stage_rows = _SPM_PAD // ns

    mesh = plsc.VectorSubcoreMesh(core_axis_name="c", subcore_axis_name="s")
    out_plane = jax.ShapeDtypeStruct((npts_pad,), jnp.float32)

    @functools.partial(
        pl.kernel,
        mesh=mesh,
        out_type=tuple(out_plane for _ in range(6)),
        compiler_params=pltpu.CompilerParams(
            use_tc_tiling_on_sc=False, needs_layout_passes=False),
        scratch_types=[
            pltpu.VMEM((_SMALL_PAD,), jnp.float32),
            pltpu.VMEM((_SMALL_PAD,), jnp.float32),
            pltpu.VMEM((_SMALL_PAD,), jnp.float32),
            [pltpu.VMEM_SHARED((_SPM_PAD,), jnp.float32)
             for _ in range(_FEAT)],
            pltpu.VMEM((2 * _B,), jnp.float32),
            pltpu.VMEM((2 * _B,), jnp.float32),
            pltpu.SemaphoreType.DMA,
            [[pltpu.VMEM((_B,), jnp.int32) for _ in range(4)]
             for _ in range(_N_STREAM)],
            pltpu.VMEM((_N_STREAM, 4, _B), jnp.float32),
            [[[pltpu.VMEM((_B,), jnp.float32) for _ in range(4)]
              for _ in range(_FEAT)]
             for _ in range(_N_STREAM)],
            pltpu.SemaphoreType.DMA,
            pltpu.VMEM((3, _B), jnp.float32),
            [[pltpu.VMEM((_B,), jnp.float32) for _ in range(3)]
             for _ in range(2)],
            pltpu.SemaphoreType.DMA,
        ],
    )
    def grid_kernel(pts_hbm, s0_hbm, s1_hbm, s2_hbm,
                    sp00, sp01, sp02, sp10, sp11, sp12,
                    o00, o01, o02, o10, o11, o12,
                    s0, s1, s2, spm, xya, xyb, in_sem,
                    idxb, wb, rows, gsem, acc, ostage, osem):
        cid = lax.axis_index("c")
        sid = lax.axis_index("s")
        base_pt = sid * pts_per_tile

        pltpu.sync_copy(s0_hbm, s0)
        pltpu.sync_copy(s1_hbm, s1)
        pltpu.sync_copy(s2_hbm, s2)
        sl = pl.ds(sid * stage_rows, stage_rows)

        @pl.when(cid == 0)
        def _stage0():
            for src_hbm, dst in zip((sp00, sp01, sp02), spm):
                pltpu.sync_copy(src_hbm.at[sl], dst.at[sl])

        @pl.when(cid == 1)
        def _stage1():
            for src_hbm, dst in zip((sp10, sp11, sp12), spm):
                pltpu.sync_copy(src_hbm.at[sl], dst.at[sl])

        plsc.subcore_barrier()

        i16 = lax.iota(jnp.int32, 16)

        pltpu.make_async_copy(
            pts_hbm.at[pl.ds(2 * base_pt, 2 * _B)], xya, in_sem).start()

        def do_chunk(core, ci, xy, buf):
            outs = (o00, o01, o02) if core == 0 else (o10, o11, o12)
            sml = _CORE_SMALL[core]
            stream_lods = _CORE_STREAM[core]
            sbases = _stream_bases(core)
            start = base_pt + ci * _B
            pltpu.make_async_copy(
                pts_hbm.at[pl.ds(2 * start, 2 * _B)], xy, in_sem).wait()

            @pl.when(ci + 1 < chunks)
            def _prefetch():
                nxt = xyb if buf == 0 else xya
                pltpu.make_async_copy(
                    pts_hbm.at[pl.ds(2 * (start + _B), 2 * _B)],
                    nxt, in_sem).start()

            def group_body(g, _):
                gofs = g * 16
                rowi2 = 2 * (i16 + gofs)
                xs = plsc.load_gather(xy, [rowi2])
                ys = plsc.load_gather(xy, [rowi2 + 1])
                a0 = jnp.zeros((16,), jnp.float32)
                a1 = jnp.zeros((16,), jnp.float32)
                a2 = jnp.zeros((16,), jnp.float32)
                for l in sml:
                    res = _LODS[l]
                    cx, cy, w = _coords(xs, ys, res)
                    idx4 = _corner_indices(cx, cy, res, _SIZES[l],
                                           _SMALL_BASES[l])
                    for wgt, idx in zip(w, idx4):
                        a0 = a0 + wgt * plsc.load_gather(s0, [idx])
                        a1 = a1 + wgt * plsc.load_gather(s1, [idx])
                        a2 = a2 + wgt * plsc.load_gather(s2, [idx])
                acc[0, pl.ds(gofs, 16)] = a0
                acc[1, pl.ds(gofs, 16)] = a1
                acc[2, pl.ds(gofs, 16)] = a2
                for li, l in enumerate(stream_lods):
                    res = _LODS[l]
                    cx, cy, w = _coords(xs, ys, res)
                    idx4 = _corner_indices(cx, cy, res, _SIZES[l],
                                           sbases[li])
                    for c in range(4):
                        idxb[li][c][pl.ds(gofs, 16)] = idx4[c]
                        wb[li, c, pl.ds(gofs, 16)] = w[c]
                return 0

            lax.fori_loop(0, _G, group_body, 0)

            descs = []
            for li in range(_N_STREAM):
                for c in range(4):
                    for f in range(_FEAT):
                        d = pltpu.make_async_copy(
                            spm[f].at[idxb[li][c]], rows[li][f][c], gsem)
                        d.start()
                        descs.append(d)
            for d in descs:
                d.wait()

            def combine_body(g, _):
                gofs = g * 16
                sl16 = pl.ds(gofs, 16)
                a0 = acc[0, sl16]
                a1 = acc[1, sl16]
                a2 = acc[2, sl16]
                for li in range(_N_STREAM):
                    for c in range(4):
                        wgt = wb[li, c, sl16]
                        a0 = a0 + wgt * rows[li][0][c][sl16]
                        a1 = a1 + wgt * rows[li][1][c][sl16]
                        a2 = a2 + wgt * rows[li][2][c][sl16]
                ostage[buf][0][sl16] = a0
                ostage[buf][1][sl16] = a1
                ostage[buf][2][sl16] = a2
                return 0

            lax.fori_loop(0, _G, combine_body, 0)

            @pl.when(ci >= 2)
            def _reclaim():
                for f in range(3):
                    pltpu.make_async_copy(
                        ostage[buf][f],
                        outs[f].at[pl.ds(start, _B)], osem).wait()

            for f in range(3):
                pltpu.make_async_copy(
                    ostage[buf][f],
                    outs[f].at[pl.ds(start, _B)], osem).start()

        def core_work(core):
            def outer_body(co, _):
                do_chunk(core, 2 * co, xya, 0)
                do_chunk(core, 2 * co + 1, xyb, 1)
                return 0

            lax.fori_loop(0, chunks // 2, outer_body, 0)

        @pl.when(cid == 0)
        def _work0():
            core_work(0)

        @pl.when(cid == 1)
        def _work1():
            core_work(1)

        for _ in range(2):
            for f in range(3):
                pltpu.make_async_copy(
                    ostage[0][f], o00.at[pl.ds(0, _B)], osem).wait()

    return grid_kernel


def _combine_kernel(a0, b0, a1, b1, a2, b2, y0, y1, y2):
    y0[...] = a0[...] + b0[...]
    y1[...] = a1[...] + b1[...]
    y2[...] = a2[...] + b2[...]


def _combine(planes0, planes1, npts_pad):
    cols = 1024
    rows = npts_pad // cols
    br = 128 if rows % 128 == 0 else rows
    spec = pl.BlockSpec((br, cols), lambda i: (i, 0))
    out2d = jax.ShapeDtypeStruct((rows, cols), jnp.float32)
    args = []
    for p0, p1 in zip(planes0, planes1):
        args += [p0.reshape(rows, cols), p1.reshape(rows, cols)]
    outs = pl.pallas_call(
        _combine_kernel,
        grid=(rows // br,),
        in_specs=[spec] * 6,
        out_specs=[spec] * 3,
        out_shape=[out2d] * 3,
    )(*args)
    return [o.reshape(-1) for o in outs]


def kernel(pts, table_0, table_1, table_2, table_3, table_4, table_5,
           table_6, table_7, table_8, table_9, table_10, table_11,
           table_12, table_13, table_14, table_15):
    tables = [table_0, table_1, table_2, table_3, table_4, table_5,
              table_6, table_7, table_8, table_9, table_10, table_11,
              table_12, table_13, table_14, table_15]
    n = pts.shape[0]
    npts_pad = ((n + 4095) // 4096) * 4096

    small = jnp.concatenate(tables[:_N_SMALL], axis=0)
    pad_s = _SMALL_PAD - _SMALL_TOTAL
    splanes = [jnp.pad(small[:, f], (0, pad_s)) for f in range(_FEAT)]
    spms = []
    for core in range(2):
        t = jnp.concatenate([tables[l] for l in _CORE_STREAM[core]], axis=0)
        for f in range(_FEAT):
            spms.append(jnp.pad(t[:, f], (0, _SPM_PAD - t.shape[0])))
    pts_flat = jnp.pad(pts, ((0, npts_pad - n), (0, 0))).reshape(-1)

    parts = _make_sc_kernel(npts_pad)(pts_flat, *splanes, *spms)
    planes = _combine(parts[:3], parts[3:], npts_pad)
    return jnp.stack([p[:n] for p in planes], axis=-1)


# speedup vs baseline: 1.1077x; 1.1077x over previous
import functools

import jax
import jax.numpy as jnp
import numpy as np
from jax import lax
from jax.experimental import pallas as pl
from jax.experimental.pallas import tpu as pltpu
from jax.experimental.pallas import tpu_sc as plsc

_MIN_RES = 16
_MAX_RES = 512
_NUM_LOD = 16
_CODEBOOK = 2 ** 16
_FEAT = 3
_bexp = np.exp((np.log(_MAX_RES) - np.log(_MIN_RES)) / (_NUM_LOD - 1))
_LODS = [int(1 + np.floor(_MIN_RES * _bexp ** l)) for l in range(_NUM_LOD)]
_SIZES = [min(l * l, _CODEBOOK) for l in _LODS]
_PRIME = np.uint32(2654435761)

_N_SMALL = 6
_SMALL_SIZES = _SIZES[:_N_SMALL]
_SMALL_BASES = np.concatenate(([0], np.cumsum(_SMALL_SIZES)))[:-1].tolist()
_SMALL_TOTAL = int(np.sum(_SMALL_SIZES))
_SMALL_PAD = ((_SMALL_TOTAL + 63) // 64) * 64

_CORE_SMALL = ([0, 1, 2], [3, 4, 5])
_CORE_STREAM = ([6, 7, 11, 12, 13], [8, 9, 10, 14, 15])
_N_STREAM = 5
_SPM_ROWS = max(sum(_SIZES[l] for l in s) for s in _CORE_STREAM)
_SPM_PAD = ((_SPM_ROWS + 127) // 128) * 128


def _stream_bases(core):
    bases, acc = [], 0
    for l in _CORE_STREAM[core]:
        bases.append(acc)
        acc += _SIZES[l]
    return bases


_B = 128
_G = _B // 16


def _corner_indices(cx, cy, res, size, base):
    if res * res <= size:
        i00 = cy * res + (cx + base)
        i10 = i00 + 1
        i01 = i00 + res
        i11 = i01 + 1
    else:
        ux = cx.astype(jnp.uint32)
        uy = cy.astype(jnp.uint32)
        ux1 = ux + jnp.uint32(1)
        hy0 = uy * _PRIME
        hy1 = hy0 + _PRIME
        m = jnp.uint32(size - 1)
        i00 = (( ux ^ hy0) & m).astype(jnp.int32) + base
        i10 = ((ux1 ^ hy0) & m).astype(jnp.int32) + base
        i01 = (( ux ^ hy1) & m).astype(jnp.int32) + base
        i11 = ((ux1 ^ hy1) & m).astype(jnp.int32) + base
    return i00, i10, i01, i11


def _coords(xs, ys, res):
    sx = xs * np.float32(res - 1)
    sy = ys * np.float32(res - 1)
    cx = jnp.minimum(sx.astype(jnp.int32), res - 2)
    cy = jnp.minimum(sy.astype(jnp.int32), res - 2)
    fx = sx - cx.astype(jnp.float32)
    fy = sy - cy.astype(jnp.float32)
    gx = 1.0 - fx
    gy = 1.0 - fy
    w = (gx * gy, fx * gy, gx * fy, fx * fy)
    return cx, cy, w


def _make_sc_kernel(npts_pad):
    info = plsc.get_sparse_core_info()
    nc, ns = info.num_cores, info.num_subcores
    pts_per_tile = npts_pad // ns
    chunks = pts_per_tile // _B
    stage_rows = _SPM_PAD // ns

    mesh = plsc.VectorSubcoreMesh(core_axis_name="c", subcore_axis_name="s")
    out_plane = jax.ShapeDtypeStruct((npts_pad,), jnp.float32)

    @functools.partial(
        pl.kernel,
        mesh=mesh,
        out_type=tuple(out_plane for _ in range(6)),
        compiler_params=pltpu.CompilerParams(
            use_tc_tiling_on_sc=False, needs_layout_passes=False),
        scratch_types=[
            pltpu.VMEM((_SMALL_PAD,), jnp.float32),
            pltpu.VMEM((_SMALL_PAD,), jnp.float32),
            pltpu.VMEM((_SMALL_PAD,), jnp.float32),
            [pltpu.VMEM_SHARED((_SPM_PAD,), jnp.float32)
             for _ in range(_FEAT)],
            pltpu.VMEM((2 * _B,), jnp.float32),
            pltpu.VMEM((2 * _B,), jnp.float32),
            pltpu.SemaphoreType.DMA,
            [[[pltpu.VMEM((_B,), jnp.int32) for _ in range(4)]
              for _ in range(_N_STREAM)] for _ in range(2)],
            pltpu.VMEM((2, _N_STREAM, 4, _B), jnp.float32),
            [[[[pltpu.VMEM((_B,), jnp.float32) for _ in range(4)]
               for _ in range(_FEAT)]
              for _ in range(_N_STREAM)] for _ in range(2)],
            pltpu.SemaphoreType.DMA,
            pltpu.VMEM((2, 3, _B), jnp.float32),
            [[pltpu.VMEM((_B,), jnp.float32) for _ in range(3)]
             for _ in range(2)],
            pltpu.SemaphoreType.DMA,
        ],
    )
    def grid_kernel(pts_hbm, s0_hbm, s1_hbm, s2_hbm,
                    sp00, sp01, sp02, sp10, sp11, sp12,
                    o00, o01, o02, o10, o11, o12,
                    s0, s1, s2, spm, xya, xyb, in_sem,
                    idxb, wb, rows, gsem, acc, ostage, osem):
        cid = lax.axis_index("c")
        sid = lax.axis_index("s")
        base_pt = sid * pts_per_tile

        pltpu.sync_copy(s0_hbm, s0)
        pltpu.sync_copy(s1_hbm, s1)
        pltpu.sync_copy(s2_hbm, s2)
        sl = pl.ds(sid * stage_rows, stage_rows)

        @pl.when(cid == 0)
        def _stage0():
            for src_hbm, dst in zip((sp00, sp01, sp02), spm):
                pltpu.sync_copy(src_hbm.at[sl], dst.at[sl])

        @pl.when(cid == 1)
        def _stage1():
            for src_hbm, dst in zip((sp10, sp11, sp12), spm):
                pltpu.sync_copy(src_hbm.at[sl], dst.at[sl])

        plsc.subcore_barrier()

        i16 = lax.iota(jnp.int32, 16)

        pltpu.make_async_copy(
            pts_hbm.at[pl.ds(2 * base_pt, 2 * _B)], xya, in_sem).start()

        def front(core, ci, xy, p):
            sml = _CORE_SMALL[core]
            stream_lods = _CORE_STREAM[core]
            sbases = _stream_bases(core)
            start = base_pt + ci * _B
            pltpu.make_async_copy(
                pts_hbm.at[pl.ds(2 * start, 2 * _B)], xy, in_sem).wait()

            @pl.when(ci + 1 < chunks)
            def _prefetch():
                nxt = xyb if p == 0 else xya
                pltpu.make_async_copy(
                    pts_hbm.at[pl.ds(2 * (start + _B), 2 * _B)],
                    nxt, in_sem).start()

            def group_body(g, _):
                gofs = g * 16
                rowi2 = 2 * (i16 + gofs)
                xs = plsc.load_gather(xy, [rowi2])
                ys = plsc.load_gather(xy, [rowi2 + 1])
                a0 = jnp.zeros((16,), jnp.float32)
                a1 = jnp.zeros((16,), jnp.float32)
                a2 = jnp.zeros((16,), jnp.float32)
                for l in sml:
                    res = _LODS[l]
                    cx, cy, w = _coords(xs, ys, res)
                    idx4 = _corner_indices(cx, cy, res, _SIZES[l],
                                           _SMALL_BASES[l])
                    for wgt, idx in zip(w, idx4):
                        a0 = a0 + wgt * plsc.load_gather(s0, [idx])
                        a1 = a1 + wgt * plsc.load_gather(s1, [idx])
                        a2 = a2 + wgt * plsc.load_gather(s2, [idx])
                acc[p, 0, pl.ds(gofs, 16)] = a0
                acc[p, 1, pl.ds(gofs, 16)] = a1
                acc[p, 2, pl.ds(gofs, 16)] = a2
                for li, l in enumerate(stream_lods):
                    res = _LODS[l]
                    cx, cy, w = _coords(xs, ys, res)
                    idx4 = _corner_indices(cx, cy, res, _SIZES[l],
                                           sbases[li])
                    for c in range(4):
                        idxb[p][li][c][pl.ds(gofs, 16)] = idx4[c]
                        wb[p, li, c, pl.ds(gofs, 16)] = w[c]
                return 0

            lax.fori_loop(0, _G, group_body, 0)

        def stream_descs(p):
            return [pltpu.make_async_copy(
                        spm[f].at[idxb[p][li][c]], rows[p][li][f][c], gsem)
                    for li in range(_N_STREAM)
                    for c in range(4)
                    for f in range(_FEAT)]

        def fire(p):
            for d in stream_descs(p):
                d.start()

        def back(core, ci, p):
            outs = (o00, o01, o02) if core == 0 else (o10, o11, o12)
            start = base_pt + ci * _B
            for d in stream_descs(p):
                d.wait()

            def combine_body(g, _):
                gofs = g * 16
                sl16 = pl.ds(gofs, 16)
                a0 = acc[p, 0, sl16]
                a1 = acc[p, 1, sl16]
                a2 = acc[p, 2, sl16]
                for li in range(_N_STREAM):
                    for c in range(4):
                        wgt = wb[p, li, c, sl16]
                        a0 = a0 + wgt * rows[p][li][0][c][sl16]
                        a1 = a1 + wgt * rows[p][li][1][c][sl16]
                        a2 = a2 + wgt * rows[p][li][2][c][sl16]
                ostage[p][0][sl16] = a0
                ostage[p][1][sl16] = a1
                ostage[p][2][sl16] = a2
                return 0

            lax.fori_loop(0, _G, combine_body, 0)

            @pl.when(ci >= 2)
            def _reclaim():
                for f in range(3):
                    pltpu.make_async_copy(
                        ostage[p][f],
                        outs[f].at[pl.ds(start, _B)], osem).wait()

            for f in range(3):
                pltpu.make_async_copy(
                    ostage[p][f],
                    outs[f].at[pl.ds(start, _B)], osem).start()

        def core_work(core):
            front(core, 0, xya, 0)
            fire(0)

            def outer_body(co, _):
                ci = 2 * co
                front(core, ci + 1, xyb, 1)
                back(core, ci, 0)
                fire(1)

                @pl.when(ci + 2 < chunks)
                def _next_front():
                    front(core, ci + 2, xya, 0)

                back(core, ci + 1, 1)

                @pl.when(ci + 2 < chunks)
                def _next_fire():
                    fire(0)

                return 0

            lax.fori_loop(0, chunks // 2, outer_body, 0)

        @pl.when(cid == 0)
        def _work0():
            core_work(0)

        @pl.when(cid == 1)
        def _work1():
            core_work(1)

        for _ in range(2):
            for f in range(3):
                pltpu.make_async_copy(
                    ostage[0][f], o00.at[pl.ds(0, _B)], osem).wait()

    return grid_kernel


def _combine_kernel(a0, b0, a1, b1, a2, b2, y0, y1, y2):
    y0[...] = a0[...] + b0[...]
    y1[...] = a1[...] + b1[...]
    y2[...] = a2[...] + b2[...]


def _combine(planes0, planes1, npts_pad):
    cols = 1024
    rows = npts_pad // cols
    br = 128 if rows % 128 == 0 else rows
    spec = pl.BlockSpec((br, cols), lambda i: (i, 0))
    out2d = jax.ShapeDtypeStruct((rows, cols), jnp.float32)
    args = []
    for p0, p1 in zip(planes0, planes1):
        args += [p0.reshape(rows, cols), p1.reshape(rows, cols)]
    outs = pl.pallas_call(
        _combine_kernel,
        grid=(rows // br,),
        in_specs=[spec] * 6,
        out_specs=[spec] * 3,
        out_shape=[out2d] * 3,
    )(*args)
    return [o.reshape(-1) for o in outs]


def kernel(pts, table_0, table_1, table_2, table_3, table_4, table_5,
           table_6, table_7, table_8, table_9, table_10, table_11,
           table_12, table_13, table_14, table_15):
    tables = [table_0, table_1, table_2, table_3, table_4, table_5,
              table_6, table_7, table_8, table_9, table_10, table_11,
              table_12, table_13, table_14, table_15]
    n = pts.shape[0]
    npts_pad = ((n + 4095) // 4096) * 4096

    small = jnp.concatenate(tables[:_N_SMALL], axis=0)
    pad_s = _SMALL_PAD - _SMALL_TOTAL
    splanes = [jnp.pad(small[:, f], (0, pad_s)) for f in range(_FEAT)]
    spms = []
    for core in range(2):
        t = jnp.concatenate([tables[l] for l in _CORE_STREAM[core]], axis=0)
        for f in range(_FEAT):
            spms.append(jnp.pad(t[:, f], (0, _SPM_PAD - t.shape[0])))
    pts_flat = jnp.pad(pts, ((0, npts_pad - n), (0, 0))).reshape(-1)

    parts = _make_sc_kernel(npts_pad)(pts_flat, *splanes, *spms)
    planes = _combine(parts[:3], parts[3:], npts_pad)
    return jnp.stack([p[:n] for p in planes], axis=-1)
